# Initial kernel scaffold; baseline (speedup 1.0000x reference)
import jax
import jax.numpy as jnp
from jax.experimental import pallas as pl


def kernel(x, adj, type_index, non_zero_index, non_zero_value, W1, b1, W2, b2, Wf, bf):
    raise NotImplementedError("write your pallas kernel here")


# speedup vs baseline: 1.1845x; 1.1845x over previous
import functools

import jax
import jax.numpy as jnp
from jax.experimental import pallas as pl
from jax.experimental.pallas import tpu as pltpu

_N = 10000
_BM1 = 256
_BR = 256


def _pass1_kernel(adj_ref, x_ref, W1_ref, b1_ref, W2_ref, Wf_ref,
                  z2_ref, z1_s, w2f_s):
    @pl.when(pl.program_id(0) == 0)
    def _():
        z1_s[...] = jnp.dot(x_ref[...], W1_ref[...],
                            preferred_element_type=jnp.float32)
        w2f_s[...] = jnp.dot(W2_ref[...], Wf_ref[...],
                             preferred_element_type=jnp.float32)
    t = jnp.dot(adj_ref[...], z1_s[...], preferred_element_type=jnp.float32)
    h = jnp.maximum(t + b1_ref[...], 0.0)
    z2_ref[...] = jnp.dot(h, w2f_s[...], preferred_element_type=jnp.float32)


def _pass2_kernel(ti_ref, adj_hbm, z2_ref, b2_ref, Wf_ref, bf_ref,
                  out_ref, gath_s, sem):
    step = pl.program_id(0)

    def issue(r, carry):
        idx = ti_ref[step * _BR + r]
        pltpu.make_async_copy(adj_hbm.at[idx], gath_s.at[r], sem).start()
        return carry

    jax.lax.fori_loop(0, _BR, issue, 0)

    def drain(r, carry):
        pltpu.make_async_copy(adj_hbm.at[0], gath_s.at[0], sem).wait()
        return carry

    jax.lax.fori_loop(0, _BR, drain, 0)

    acc = jnp.dot(gath_s[...], z2_ref[...], preferred_element_type=jnp.float32)
    bias = jnp.dot(b2_ref[...], Wf_ref[...],
                   preferred_element_type=jnp.float32) + bf_ref[...]
    o = acc + bias
    m = jnp.max(o, axis=1, keepdims=True)
    lse = m + jnp.log(jnp.sum(jnp.exp(o - m), axis=1, keepdims=True))
    out_ref[...] = o - lse


def kernel(x, adj, type_index, non_zero_index, non_zero_value,
           W1, b1, W2, b2, Wf, bf):
    n, nfeat = x.shape
    nhid2 = W1.shape[1]
    nhid = W2.shape[1]
    ncls = Wf.shape[1]
    t = type_index.shape[0]

    b1r = b1.reshape(1, nhid2)
    b2r = b2.reshape(1, nhid)
    bfr = bf.reshape(1, ncls)

    z2 = pl.pallas_call(
        _pass1_kernel,
        grid=(pl.cdiv(n, _BM1),),
        in_specs=[
            pl.BlockSpec((_BM1, n), lambda i: (i, 0)),
            pl.BlockSpec((n, nfeat), lambda i: (0, 0)),
            pl.BlockSpec((nfeat, nhid2), lambda i: (0, 0)),
            pl.BlockSpec((1, nhid2), lambda i: (0, 0)),
            pl.BlockSpec((nhid2, nhid), lambda i: (0, 0)),
            pl.BlockSpec((nhid, ncls), lambda i: (0, 0)),
        ],
        out_specs=pl.BlockSpec((_BM1, ncls), lambda i: (i, 0)),
        out_shape=jax.ShapeDtypeStruct((n, ncls), jnp.float32),
        scratch_shapes=[pltpu.VMEM((n, nhid2), jnp.float32),
                        pltpu.VMEM((nhid2, ncls), jnp.float32)],
    )(adj, x, W1, b1r, W2, Wf)

    grid_spec = pltpu.PrefetchScalarGridSpec(
        num_scalar_prefetch=1,
        grid=(t // _BR,),
        in_specs=[
            pl.BlockSpec(memory_space=pl.ANY),
            pl.BlockSpec((n, ncls), lambda i, ti: (0, 0)),
            pl.BlockSpec((1, nhid), lambda i, ti: (0, 0)),
            pl.BlockSpec((nhid, ncls), lambda i, ti: (0, 0)),
            pl.BlockSpec((1, ncls), lambda i, ti: (0, 0)),
        ],
        out_specs=pl.BlockSpec((_BR, ncls), lambda i, ti: (i, 0)),
        scratch_shapes=[pltpu.VMEM((_BR, n), jnp.float32),
                        pltpu.SemaphoreType.DMA],
    )
    out = pl.pallas_call(
        _pass2_kernel,
        grid_spec=grid_spec,
        out_shape=jax.ShapeDtypeStruct((t, ncls), jnp.float32),
    )(type_index, adj, z2, b2r, Wf, bfr)
    return out
